# baseline (device time: 22280 ns/iter reference)
import jax
import jax.numpy as jnp
from jax import lax
from jax.experimental import pallas as pl
from jax.experimental.pallas import tpu as pltpu

N_DEV = 8


def kernel(x, w_mat):
    m_global, k_per = x.shape
    k_global, n = w_mat.shape
    m_per = m_global // N_DEV

    def body(x_ref, w_ref, out_ref, xs_ref, xt_ref, send_sems, recv_sems):
        my = lax.axis_index("i")

        barrier_sem = pltpu.get_barrier_semaphore()
        for d in range(1, N_DEV):
            pl.semaphore_signal(
                barrier_sem, inc=1,
                device_id=((my + d) % N_DEV,),
                device_id_type=pl.DeviceIdType.MESH,
            )
        pl.semaphore_wait(barrier_sem, N_DEV - 1)

        rdmas = []
        for d in range(1, N_DEV):
            dst = (my + d) % N_DEV
            xs_ref[d] = x_ref[pl.ds(dst * m_per, m_per), :].astype(jnp.bfloat16)
            rdma = pltpu.make_async_remote_copy(
                src_ref=xs_ref.at[d],
                dst_ref=xt_ref.at[d],
                send_sem=send_sems.at[d],
                recv_sem=recv_sems.at[d],
                device_id=(dst,),
                device_id_type=pl.DeviceIdType.MESH,
            )
            rdma.start()
            rdmas.append(rdma)

        own = x_ref[pl.ds(my * m_per, m_per), :].astype(jnp.bfloat16)
        w_own = w_ref[pl.ds(my * k_per, k_per), :].astype(jnp.bfloat16)
        out_ref[...] = jnp.dot(own, w_own, preferred_element_type=jnp.float32)

        for d in range(1, N_DEV):
            rdmas[d - 1].wait()
            src = (my - d) % N_DEV
            w_src = w_ref[pl.ds(src * k_per, k_per), :].astype(jnp.bfloat16)
            out_ref[...] += jnp.dot(
                xt_ref[d], w_src, preferred_element_type=jnp.float32
            )

    return pl.pallas_call(
        body,
        out_shape=jax.ShapeDtypeStruct((m_per, n), jnp.float32),
        in_specs=[
            pl.BlockSpec(memory_space=pltpu.VMEM),
            pl.BlockSpec(memory_space=pltpu.VMEM),
        ],
        out_specs=pl.BlockSpec(memory_space=pltpu.VMEM),
        scratch_shapes=[
            pltpu.VMEM((N_DEV, m_per, k_per), jnp.bfloat16),
            pltpu.VMEM((N_DEV, m_per, k_per), jnp.bfloat16),
            pltpu.SemaphoreType.DMA((N_DEV,)),
            pltpu.SemaphoreType.DMA((N_DEV,)),
        ],
        compiler_params=pltpu.CompilerParams(collective_id=0),
    )(x, w_mat)
